# baseline (speedup 1.0000x reference)
import functools
import math

import jax
import jax.numpy as jnp
from jax import lax
from jax.experimental import pallas as pl
from jax.experimental.pallas import tpu as pltpu
from jax.experimental.pallas import tpu_sc as plsc

DIM = 64
SCALE = math.sqrt(DIM)

NC = 2
NS = 16
NW = NC * NS
LANES = 16
NB = 4
SPLIT = 128


@functools.lru_cache(maxsize=None)
def _build(n_rows: int, n_cols: int, vocab: int):
    rows_per_w = n_rows // NW
    assert rows_per_w * NW == n_rows and rows_per_w % NB == 0
    rest = n_cols - SPLIT
    assert 0 < rest <= SPLIT and SPLIT % 8 == 0

    mesh = plsc.VectorSubcoreMesh(core_axis_name="c", subcore_axis_name="s")

    scratch = (
        [pltpu.VMEM((rows_per_w, n_cols), jnp.int32)]
        + [pltpu.VMEM((n_cols, DIM), jnp.float32) for _ in range(2 * NB)]
        + [pltpu.SemaphoreType.DMA for _ in range(2 * NB)]
    )

    @functools.partial(
        pl.kernel,
        mesh=mesh,
        compiler_params=pltpu.CompilerParams(use_tc_tiling_on_sc=False),
        out_type=jax.ShapeDtypeStruct((n_rows, n_cols, DIM), jnp.float32),
        scratch_types=scratch,
    )
    def emb_kernel(tok_hbm, table_hbm, out_hbm, tok_v, *bufs):
        rows_in = bufs[:NB]
        rows_out = bufs[NB:2 * NB]
        in_sem = bufs[2 * NB:3 * NB]
        out_sem = bufs[3 * NB:]

        wid = lax.axis_index("s") * NC + lax.axis_index("c")
        wbase = wid * rows_per_w
        pltpu.sync_copy(tok_hbm.at[pl.ds(wbase, rows_per_w)], tok_v)

        def fire_gather(r, b):
            pltpu.async_copy(
                table_hbm.at[tok_v.at[r, pl.ds(0, SPLIT)]],
                rows_in[b].at[pl.ds(0, SPLIT)],
                in_sem[b],
            )
            pltpu.async_copy(
                table_hbm.at[tok_v.at[r, pl.ds(SPLIT, rest)]],
                rows_in[b].at[pl.ds(SPLIT, rest)],
                in_sem[b],
            )

        def wait_gather(r, b):
            pltpu.make_async_copy(
                table_hbm.at[tok_v.at[r, pl.ds(0, SPLIT)]],
                rows_in[b].at[pl.ds(0, SPLIT)],
                in_sem[b],
            ).wait()
            pltpu.make_async_copy(
                table_hbm.at[tok_v.at[r, pl.ds(SPLIT, rest)]],
                rows_in[b].at[pl.ds(SPLIT, rest)],
                in_sem[b],
            ).wait()

        for b in range(NB):
            fire_gather(b, b)

        @pl.loop(0, rows_per_w, step=NB)
        def row_group(g):
            for b in range(NB):
                r = g + b
                wait_gather(r, b)

                @pl.when(r >= NB)
                def _():
                    pltpu.make_async_copy(
                        rows_out[b], out_hbm.at[wbase], out_sem[b]
                    ).wait()

                src = rows_in[b]
                dst = rows_out[b]

                @plsc.parallel_loop(0, n_cols, unroll=8)
                def scale_row(t):
                    for c in range(DIM // LANES):
                        sl = pl.ds(c * LANES, LANES)
                        dst[t, sl] = src[t, sl] * SCALE

                pltpu.async_copy(dst, out_hbm.at[wbase + r], out_sem[b])

                @pl.when(r + NB < rows_per_w)
                def _():
                    fire_gather(r + NB, b)

        for b in range(NB):
            pltpu.make_async_copy(
                rows_out[b], out_hbm.at[wbase], out_sem[b]
            ).wait()

    return emb_kernel


def kernel(tokens, W):
    n_rows, n_cols = tokens.shape
    out = _build(n_rows, n_cols, W.shape[0])(tokens.astype(jnp.int32), W)
    return out
